# baseline (speedup 1.0000x reference)
import functools

import jax
import jax.numpy as jnp
from jax import lax
from jax.experimental import pallas as pl
from jax.experimental.pallas import tpu as pltpu
from jax.experimental.pallas import tpu_sc as plsc

N_NODES = 10000
N_EDGES = 320000
N_GRAPHS = 16
D = 128

NW = 32
RPT = 320
NPAD = NW * RPT
CHUNK = 8000
NCHUNK = N_EDGES // CHUNK
SUB = 128
BR = 1000
NEG_INF = float("-inf")


def _segmax_body(x_hbm, src_hbm, dst_hbm, out_hbm,
                 src_stage, dst_stage, msrc, mdst, rows_buf, acc, sem):
    wid = lax.axis_index("s") * 2 + lax.axis_index("c")
    lo = wid * RPT

    def init_row(r, carry):
        for k in range(D // 16):
            acc[r, pl.ds(k * 16, 16)] = jnp.full((16,), NEG_INF, jnp.float32)
        return carry
    lax.fori_loop(0, RPT + 1, init_row, 0)

    lane = jnp.arange(16, dtype=jnp.int32)
    lov = jnp.broadcast_to(lo, (16,))
    hiv = jnp.broadcast_to(lo + RPT, (16,))

    def chunk_body(c, carry):
        base = pl.multiple_of(c * CHUNK, 256)
        pltpu.sync_copy(src_hbm.at[pl.ds(base, CHUNK)], src_stage)
        pltpu.sync_copy(dst_hbm.at[pl.ds(base, CHUNK)], dst_stage)

        def scan_body(j, cnt):
            s = src_stage[pl.ds(j * 16, 16)]
            d = dst_stage[pl.ds(j * 16, 16)]
            m = (d >= lov) & (d < hiv)
            mi = jnp.where(m, jnp.full((16,), 1, jnp.int32),
                           jnp.zeros((16,), jnp.int32))
            pos = jnp.broadcast_to(cnt - 1, (16,)) + plsc.cumsum(mi)
            plsc.store_scatter(msrc, [pos], s, mask=m)
            plsc.store_scatter(mdst, [pos], d - lov, mask=m)
            return cnt + jnp.sum(mi)
        cnt = lax.fori_loop(0, CHUNK // 16, scan_body, jnp.int32(0))

        npad = (SUB - cnt % SUB) % SUB
        total = cnt + npad

        def pad_body(p, carry):
            idxs = jnp.broadcast_to(cnt + p * 16, (16,)) + lane
            pm = idxs < jnp.broadcast_to(total, (16,))
            plsc.store_scatter(msrc, [idxs],
                               jnp.zeros((16,), jnp.int32), mask=pm)
            plsc.store_scatter(mdst, [idxs],
                               jnp.full((16,), RPT, jnp.int32), mask=pm)
            return carry
        lax.fori_loop(0, (npad + 15) // 16, pad_body, 0)

        def sub_body(si, carry):
            sbase = pl.multiple_of(si * SUB, SUB)
            pltpu.async_copy(x_hbm.at[msrc.at[pl.ds(sbase, SUB)]],
                             rows_buf, sem).wait()

            def grp_body(g, carry2):
                dv = mdst[pl.ds(sbase + g * 16, 16)]
                for e in range(16):
                    dl = dv[e]
                    r = g * 16 + e
                    for k in range(D // 16):
                        v = rows_buf[r, pl.ds(k * 16, 16)]
                        a = acc[dl, pl.ds(k * 16, 16)]
                        acc[dl, pl.ds(k * 16, 16)] = jnp.maximum(a, v)
                return carry2
            lax.fori_loop(0, SUB // 16, grp_body, 0)
            return carry
        lax.fori_loop(0, total // SUB, sub_body, 0)
        return carry

    lax.fori_loop(0, NCHUNK, chunk_body, 0)

    def fin_body(r, carry):
        for k in range(D // 16):
            a = acc[r, pl.ds(k * 16, 16)]
            acc[r, pl.ds(k * 16, 16)] = jnp.where(a == NEG_INF, 0.0, a)
        return carry
    lax.fori_loop(0, RPT, fin_body, 0)
    pltpu.sync_copy(acc.at[pl.ds(0, RPT)], out_hbm.at[pl.ds(lo, RPT)])


@jax.jit
def _segmax_call(x, src, dst):
    mesh = plsc.VectorSubcoreMesh(core_axis_name="c", subcore_axis_name="s",
                                  num_cores=2, num_subcores=16)
    out = pl.kernel(
        _segmax_body,
        out_type=jax.ShapeDtypeStruct((NPAD, D), jnp.float32),
        mesh=mesh,
        compiler_params=pltpu.CompilerParams(needs_layout_passes=False),
        scratch_types=[
            pltpu.VMEM((CHUNK,), jnp.int32),
            pltpu.VMEM((CHUNK,), jnp.int32),
            pltpu.VMEM((CHUNK,), jnp.int32),
            pltpu.VMEM((CHUNK,), jnp.int32),
            pltpu.VMEM((SUB, D), jnp.float32),
            pltpu.VMEM((RPT + 1, D), jnp.float32),
            pltpu.SemaphoreType.DMA,
        ],
    )(x, src, dst)
    return out[:N_NODES]


def _silu(v):
    return v / (1.0 + jnp.exp(-v))


def _stage_a_kernel(agg_ref, x_ref, wlt_ref, bl_ref, wrt_ref, y_ref):
    z = jnp.dot(agg_ref[...], wlt_ref[...], preferred_element_type=jnp.float32)
    z += jnp.dot(x_ref[...], wrt_ref[...], preferred_element_type=jnp.float32)
    z += bl_ref[...]
    y_ref[...] = _silu(z)


def _stage_a_call(agg, x, WlT, bl2, WrT):
    block_rows = BR
    grid = (N_NODES // block_rows,)
    full = lambda i: (0, 0)
    return pl.pallas_call(
        _stage_a_kernel,
        grid=grid,
        in_specs=[
            pl.BlockSpec((block_rows, D), lambda i: (i, 0)),
            pl.BlockSpec((block_rows, D), lambda i: (i, 0)),
            pl.BlockSpec((D, D), full),
            pl.BlockSpec((1, D), full),
            pl.BlockSpec((D, D), full),
        ],
        out_specs=pl.BlockSpec((block_rows, D), lambda i: (i, 0)),
        out_shape=jax.ShapeDtypeStruct((N_NODES, D), jnp.float32),
    )(agg, x, WlT, bl2, WrT)


def _stage_b_kernel(agg2_ref, y1_ref, x_ref, batch_ref,
                    wlt_ref, bl_ref, wrt_ref,
                    w1t_ref, b1_ref, gamma_ref, beta_ref,
                    w2t_ref, b2_ref, wrot_ref, bro_ref,
                    out_ref, sums_ref, counts_ref, maxs_ref):
    i = pl.program_id(0)
    nsteps = pl.num_programs(0)

    x2 = jnp.dot(agg2_ref[...], wlt_ref[...], preferred_element_type=jnp.float32)
    x2 += jnp.dot(y1_ref[...], wrt_ref[...], preferred_element_type=jnp.float32)
    x2 += bl_ref[...]
    h_in = jnp.concatenate([_silu(x2), y1_ref[...], x_ref[...]], axis=1)
    h = _silu(jnp.dot(h_in, w1t_ref[...], preferred_element_type=jnp.float32)
              + b1_ref[...])
    mu = jnp.mean(h, axis=-1, keepdims=True)
    var = jnp.mean((h - mu) ** 2, axis=-1, keepdims=True)
    h = (h - mu) / jnp.sqrt(var + 1e-5) * gamma_ref[...] + beta_ref[...]
    o = jnp.dot(h, w2t_ref[...], preferred_element_type=jnp.float32) + b2_ref[...]

    bb = batch_ref[...]
    br = bb.shape[0]
    g_ids = lax.broadcasted_iota(jnp.int32, (br, N_GRAPHS), 1)
    ohb = bb == g_ids
    ohf = ohb.astype(jnp.float32)

    @pl.when(i == 0)
    def _():
        sums_ref[...] = jnp.zeros_like(sums_ref)
        counts_ref[...] = jnp.zeros_like(counts_ref)
        maxs_ref[...] = jnp.full_like(maxs_ref, NEG_INF)
        out_ref[...] = jnp.zeros_like(out_ref)

    dn = (((0,), (0,)), ((), ()))
    sums_ref[...] += lax.dot_general(ohf, o, dn,
                                     preferred_element_type=jnp.float32)
    counts_ref[...] += lax.dot_general(
        ohf, jnp.ones((br, D), jnp.float32), dn,
        preferred_element_type=jnp.float32)
    for g in range(N_GRAPHS):
        mg = jnp.max(jnp.where(ohb[:, g:g + 1], o, NEG_INF),
                     axis=0, keepdims=True)
        maxs_ref[pl.ds(g, 1), :] = jnp.maximum(maxs_ref[pl.ds(g, 1), :], mg)

    @pl.when(i == nsteps - 1)
    def _():
        mean_p = sums_ref[...] / jnp.maximum(counts_ref[...], 1.0)
        maxs = maxs_ref[...]
        max_p = jnp.where(maxs == NEG_INF, 0.0, maxs)
        pooled = jnp.concatenate([mean_p, max_p], axis=1)
        out_ref[...] = (jnp.dot(pooled, wrot_ref[...],
                                preferred_element_type=jnp.float32)
                        + bro_ref[...])


def _stage_b_call(agg2, y1, x, batch2, WlT, bl2, WrT, W1T, b12, gamma2, beta2,
                  W2T, b22, WroTp, brop):
    block_rows = BR
    grid = (N_NODES // block_rows,)
    full = lambda i: (0, 0)
    return pl.pallas_call(
        _stage_b_kernel,
        grid=grid,
        in_specs=[
            pl.BlockSpec((block_rows, D), lambda i: (i, 0)),
            pl.BlockSpec((block_rows, D), lambda i: (i, 0)),
            pl.BlockSpec((block_rows, D), lambda i: (i, 0)),
            pl.BlockSpec((block_rows, 1), lambda i: (i, 0)),
            pl.BlockSpec((D, D), full),
            pl.BlockSpec((1, D), full),
            pl.BlockSpec((D, D), full),
            pl.BlockSpec((3 * D, D), full),
            pl.BlockSpec((1, D), full),
            pl.BlockSpec((1, D), full),
            pl.BlockSpec((1, D), full),
            pl.BlockSpec((D, D), full),
            pl.BlockSpec((1, D), full),
            pl.BlockSpec((2 * D, D), full),
            pl.BlockSpec((1, D), full),
        ],
        out_specs=pl.BlockSpec((N_GRAPHS, D), full),
        out_shape=jax.ShapeDtypeStruct((N_GRAPHS, D), jnp.float32),
        scratch_shapes=[
            pltpu.VMEM((N_GRAPHS, D), jnp.float32),
            pltpu.VMEM((N_GRAPHS, D), jnp.float32),
            pltpu.VMEM((N_GRAPHS, D), jnp.float32),
        ],
    )(agg2, y1, x, batch2, WlT, bl2, WrT, W1T, b12, gamma2, beta2,
      W2T, b22, WroTp, brop)


def kernel(x, edge_index, batch, Wl1, bl1, Wr1, Wl2, bl2, Wr2,
           W1, b1, gamma, beta, W2, b2, Wro, bro):
    src = edge_index[0]
    dst = edge_index[1]

    agg1 = _segmax_call(x, src, dst)
    y1 = _stage_a_call(agg1, x, Wl1.T, bl1.reshape(1, D), Wr1.T)
    agg2 = _segmax_call(y1, src, dst)

    WroTp = jnp.zeros((2 * D, D), jnp.float32).at[:, :2].set(Wro.T)
    brop = jnp.zeros((1, D), jnp.float32).at[0, :2].set(bro)
    outp = _stage_b_call(
        agg2, y1, x, batch.reshape(N_NODES, 1),
        Wl2.T, bl2.reshape(1, D), Wr2.T,
        W1.T, b1.reshape(1, D), gamma.reshape(1, D), beta.reshape(1, D),
        W2.T, b2.reshape(1, D), WroTp, brop)
    return outp[:, :2]
